# baseline (speedup 1.0000x reference)
import jax
import jax.numpy as jnp
from jax import lax
from jax.experimental import pallas as pl
from jax.experimental.pallas import tpu as pltpu
from jax.experimental.pallas import tpu_sc as plsc

NUM_FIELDS = 26
FIELD_DIM = 100000
BATCH = 16384
L = 16
NC, NS = 2, 16
NW = NC * NS
B_PER_W = BATCH // NW
N_PER_W = B_PER_W * NUM_FIELDS
CHUNK = 128
N_CHUNKS = N_PER_W // CHUNK
ROWS_PER_CHUNK = CHUNK // L
DIV26_MUL = 40330
DIV26_SHIFT = 20


def _body(x_hbm, tab_hbm, bias_hbm, out_hbm,
          idx_v, rows_v, didx_v, binit_v, bias_v, acc_sh, sem, sem2):
    cid = lax.axis_index("c")
    sid = lax.axis_index("s")
    wid = cid * NS + sid

    pltpu.sync_copy(x_hbm.at[pl.ds(wid * N_PER_W, N_PER_W)], idx_v)
    pltpu.sync_copy(bias_hbm, bias_v)

    lane = lax.iota(jnp.int32, L)

    def add_offsets(k, carry):
        o = k * L
        idx_v[pl.ds(o, L)] = idx_v[pl.ds(o, L)] + ((lane + o) % NUM_FIELDS) * FIELD_DIM
        return carry

    lax.fori_loop(0, N_PER_W // L, add_offsets, 0)

    def fire(j, carry):
        o = j * CHUNK
        pltpu.make_async_copy(
            tab_hbm.at[idx_v.at[pl.ds(o, CHUNK)]],
            rows_v.at[pl.ds(o, CHUNK)],
            sem,
        ).start()
        return carry

    lax.fori_loop(0, N_CHUNKS, fire, 0)

    acc_base = sid * B_PER_W

    def build_didx(k, carry):
        o = k * L
        j = k // ROWS_PER_CHUNK
        col = (k % ROWS_PER_CHUNK) * L
        row = ((lane + o) * DIV26_MUL) >> DIV26_SHIFT
        didx_v[j, pl.ds(col, L)] = acc_base + row
        return carry

    lax.fori_loop(0, N_PER_W // L, build_didx, 0)

    def fill_bias(c, carry):
        binit_v[pl.ds(c * L, L)] = bias_v[...]
        return carry

    lax.fori_loop(0, B_PER_W // L, fill_bias, 0)
    pltpu.sync_copy(binit_v, acc_sh.at[pl.ds(acc_base, B_PER_W)])

    def drain(j, carry):
        o = j * CHUNK
        pltpu.make_async_copy(
            tab_hbm.at[idx_v.at[pl.ds(o, CHUNK)]],
            rows_v.at[pl.ds(o, CHUNK)],
            sem,
        ).wait()
        return carry

    lax.fori_loop(0, N_CHUNKS, drain, 0)

    def fire_scatter(j, carry):
        pltpu.async_copy(
            rows_v.at[pl.ds(j * CHUNK, CHUNK)],
            acc_sh.at[didx_v.at[j]],
            sem2,
            add=True,
        )
        return carry

    lax.fori_loop(0, N_CHUNKS, fire_scatter, 0)

    def drain_scatter(j, carry):
        pltpu.make_async_copy(
            rows_v.at[pl.ds(j * CHUNK, CHUNK)],
            acc_sh.at[didx_v.at[j]],
            sem2,
        ).wait()
        return carry

    lax.fori_loop(0, N_CHUNKS, drain_scatter, 0)

    pltpu.sync_copy(acc_sh.at[pl.ds(acc_base, B_PER_W)],
                    out_hbm.at[pl.ds(wid * B_PER_W, B_PER_W)])


@jax.jit
def _run(x_flat, tab_flat, bias16):
    mesh = plsc.VectorSubcoreMesh(core_axis_name="c", subcore_axis_name="s")
    return pl.kernel(
        _body,
        out_type=jax.ShapeDtypeStruct((BATCH,), jnp.float32),
        mesh=mesh,
        scratch_types=[
            pltpu.VMEM((N_PER_W,), jnp.int32),
            pltpu.VMEM((N_PER_W,), jnp.float32),
            pltpu.VMEM((N_CHUNKS, CHUNK), jnp.int32),
            pltpu.VMEM((B_PER_W,), jnp.float32),
            pltpu.VMEM((L,), jnp.float32),
            pltpu.VMEM_SHARED((NS * B_PER_W,), jnp.float32),
            pltpu.SemaphoreType.DMA,
            pltpu.SemaphoreType.DMA,
        ],
    )(x_flat, tab_flat, bias16)


def kernel(x, table, bias):
    x_flat = x.reshape(-1)
    tab_flat = table.reshape(-1)
    bias16 = jnp.broadcast_to(bias, (L,))
    out = _run(x_flat, tab_flat, bias16)
    return out.reshape(BATCH, 1)
